# Initial kernel scaffold; baseline (speedup 1.0000x reference)
import jax
import jax.numpy as jnp
from jax.experimental import pallas as pl


def kernel(pos_h, pos_r, pos_t, neg_h, neg_r, neg_t, entity_emb, relation_emb):
    raise NotImplementedError("write your pallas kernel here")


# speedup vs baseline: 5.1275x; 5.1275x over previous
import functools

import jax
import jax.numpy as jnp
from jax import lax
from jax.experimental import pallas as pl
from jax.experimental.pallas import tpu as pltpu
from jax.experimental.pallas import tpu_sc as plsc

L = 16
CHUNK = 128

_GATHER_DNUMS = lax.GatherDimensionNumbers(
    offset_dims=(), collapsed_slice_dims=(0,), start_index_map=(0,))


def _take16(x, idx2d):
    return lax.gather(x, idx2d, _GATHER_DNUMS, (1,),
                      mode=lax.GatherScatterMode.PROMISE_IN_BOUNDS)


def _make_sc_call(total_b, dim2):
    info = plsc.get_sparse_core_info()
    nc, ns = info.num_cores, info.num_subcores
    nw = nc * ns
    assert total_b % (nw * CHUNK) == 0
    b_per_w = total_b // nw
    n_chunks = b_per_w // CHUNK
    kpg = dim2 // L

    mesh = plsc.VectorSubcoreMesh(core_axis_name="c", subcore_axis_name="s")

    @functools.partial(
        pl.kernel,
        mesh=mesh,
        out_type=jax.ShapeDtypeStruct((total_b,), jnp.float32),
        scratch_types=[
            pltpu.VMEM((CHUNK,), jnp.int32),
            pltpu.VMEM((CHUNK,), jnp.int32),
            pltpu.VMEM((CHUNK,), jnp.int32),
            pltpu.VMEM((CHUNK, dim2), jnp.float32),
            pltpu.VMEM((CHUNK, dim2), jnp.float32),
            pltpu.VMEM((CHUNK, dim2), jnp.float32),
            pltpu.VMEM((b_per_w,), jnp.float32),
            pltpu.SemaphoreType.DMA,
        ],
    )
    def sc_call(hidx_hbm, ridx_hbm, tidx_hbm, ent_hbm, rel_hbm, out_hbm,
                hidx_v, ridx_v, tidx_v, hbuf, rbuf, tbuf, outv, sem):
        wid = lax.axis_index("s") * nc + lax.axis_index("c")
        base = wid * b_per_w

        lane = lax.iota(jnp.int32, L)
        swap2d = jnp.reshape(lane ^ 1, (L, 1))
        bfly = [jnp.reshape(lane ^ (1 << p), (L, 1)) for p in range(1, 4)]
        even = (lane & 1) == 0

        def chunk_body(ci, _):
            off = pl.multiple_of(base + ci * CHUNK, CHUNK)
            pltpu.sync_copy(hidx_hbm.at[pl.ds(off, CHUNK)], hidx_v)
            pltpu.sync_copy(ridx_hbm.at[pl.ds(off, CHUNK)], ridx_v)
            pltpu.sync_copy(tidx_hbm.at[pl.ds(off, CHUNK)], tidx_v)
            pltpu.async_copy(ent_hbm.at[hidx_v], hbuf, sem).wait()
            pltpu.async_copy(rel_hbm.at[ridx_v], rbuf, sem).wait()
            pltpu.async_copy(ent_hbm.at[tidx_v], tbuf, sem).wait()

            def group_body(gi, _):
                e0 = gi * L
                svec = jnp.zeros((L,), jnp.float32)
                for e16 in range(L):
                    e = e0 + e16
                    acc = jnp.zeros((L,), jnp.float32)
                    for k in range(kpg):
                        h = hbuf[e, pl.ds(k * L, L)]
                        t = tbuf[e, pl.ds(k * L, L)]
                        r = rbuf[e, pl.ds(k * L, L)]
                        ts = _take16(t, swap2d)
                        rs = _take16(r, swap2d)
                        rE = jnp.where(even, r, rs)
                        rOs = jnp.where(even, rs, -r)
                        acc = acc + h * (t * rE + ts * rOs)
                    acc = acc + _take16(acc, swap2d)
                    for p2d in bfly:
                        acc = acc + _take16(acc, p2d)
                    svec = jnp.where(lane == e16, acc, svec)
                outv[pl.ds(ci * CHUNK + e0, L)] = svec
                return 0

            lax.fori_loop(0, CHUNK // L, group_body, 0)
            return 0

        lax.fori_loop(0, n_chunks, chunk_body, 0)
        pltpu.sync_copy(outv, out_hbm.at[pl.ds(base, b_per_w)])

    return sc_call


def kernel(pos_h, pos_r, pos_t, neg_h, neg_r, neg_t, entity_emb, relation_emb):
    b = pos_h.shape[0]
    dim2 = entity_emb.shape[1]
    hidx = jnp.concatenate([pos_h, neg_h]).astype(jnp.int32)
    ridx = jnp.concatenate([pos_r, neg_r]).astype(jnp.int32)
    tidx = jnp.concatenate([pos_t, neg_t]).astype(jnp.int32)
    sc_call = _make_sc_call(2 * b, dim2)
    out = sc_call(hidx, ridx, tidx, entity_emb, relation_emb)
    return out[:b], out[b:]


# speedup vs baseline: 8.5493x; 1.6673x over previous
import functools

import jax
import jax.numpy as jnp
from jax import lax
from jax.experimental import pallas as pl
from jax.experimental.pallas import tpu as pltpu
from jax.experimental.pallas import tpu_sc as plsc

L = 16
CHUNK = 128

_GATHER_DNUMS = lax.GatherDimensionNumbers(
    offset_dims=(), collapsed_slice_dims=(0,), start_index_map=(0,))


def _take16(x, idx2d):
    return lax.gather(x, idx2d, _GATHER_DNUMS, (1,),
                      mode=lax.GatherScatterMode.PROMISE_IN_BOUNDS)


def _make_sc_call(total_b, dim2):
    info = plsc.get_sparse_core_info()
    nc, ns = info.num_cores, info.num_subcores
    nw = nc * ns
    assert total_b % (nw * CHUNK) == 0
    b_per_w = total_b // nw
    n_chunks = b_per_w // CHUNK
    kpg = dim2 // L

    mesh = plsc.VectorSubcoreMesh(core_axis_name="c", subcore_axis_name="s")

    assert n_chunks % 2 == 0 and n_chunks >= 4

    @functools.partial(
        pl.kernel,
        mesh=mesh,
        out_type=jax.ShapeDtypeStruct((total_b,), jnp.float32),
        scratch_types=[
            pltpu.VMEM((b_per_w,), jnp.int32),
            pltpu.VMEM((b_per_w,), jnp.int32),
            pltpu.VMEM((b_per_w,), jnp.int32),
            pltpu.VMEM((2, CHUNK, dim2), jnp.float32),
            pltpu.VMEM((2, CHUNK, dim2), jnp.float32),
            pltpu.VMEM((2, CHUNK, dim2), jnp.float32),
            pltpu.VMEM((b_per_w,), jnp.float32),
            pltpu.SemaphoreType.DMA,
            pltpu.SemaphoreType.DMA,
        ],
    )
    def sc_call(hidx_hbm, ridx_hbm, tidx_hbm, ent_hbm, rel_hbm, out_hbm,
                hidx_v, ridx_v, tidx_v, hbuf, rbuf, tbuf, outv, sem_a, sem_b):
        wid = lax.axis_index("s") * nc + lax.axis_index("c")
        base = wid * b_per_w

        lane = lax.iota(jnp.int32, L)
        swap2d = jnp.reshape(lane ^ 1, (L, 1))
        bfly = [jnp.reshape(lane ^ (1 << p), (L, 1)) for p in range(1, 4)]
        even = (lane & 1) == 0

        pltpu.sync_copy(hidx_hbm.at[pl.ds(base, b_per_w)], hidx_v)
        pltpu.sync_copy(ridx_hbm.at[pl.ds(base, b_per_w)], ridx_v)
        pltpu.sync_copy(tidx_hbm.at[pl.ds(base, b_per_w)], tidx_v)

        def copies(ci, slot, sem):
            ids = pl.ds(ci * CHUNK, CHUNK)
            return (
                pltpu.make_async_copy(ent_hbm.at[hidx_v.at[ids]], hbuf.at[slot], sem),
                pltpu.make_async_copy(rel_hbm.at[ridx_v.at[ids]], rbuf.at[slot], sem),
                pltpu.make_async_copy(ent_hbm.at[tidx_v.at[ids]], tbuf.at[slot], sem),
            )

        def start(ci, slot, sem):
            for c in copies(ci, slot, sem):
                c.start()

        def wait(ci, slot, sem):
            for c in copies(ci, slot, sem):
                c.wait()

        def compute(ci, slot):
            def group_body(gi, _):
                e0 = gi * L
                svec = jnp.zeros((L,), jnp.float32)
                for e16 in range(L):
                    e = e0 + e16
                    acc = jnp.zeros((L,), jnp.float32)
                    for k in range(kpg):
                        h = hbuf[slot, e, pl.ds(k * L, L)]
                        t = tbuf[slot, e, pl.ds(k * L, L)]
                        r = rbuf[slot, e, pl.ds(k * L, L)]
                        ts = _take16(t, swap2d)
                        rs = _take16(r, swap2d)
                        rE = jnp.where(even, r, rs)
                        rOs = jnp.where(even, rs, -r)
                        acc = acc + h * (t * rE + ts * rOs)
                    acc = acc + _take16(acc, swap2d)
                    for p2d in bfly:
                        acc = acc + _take16(acc, p2d)
                    svec = jnp.where(lane == e16, acc, svec)
                outv[pl.ds(ci * CHUNK + e0, L)] = svec
                return 0

            lax.fori_loop(0, CHUNK // L, group_body, 0)

        start(0, 0, sem_a)

        def body(i, _):
            ci_a = 2 * i
            ci_b = ci_a + 1
            start(ci_b, 1, sem_b)
            wait(ci_a, 0, sem_a)
            compute(ci_a, 0)

            @pl.when(i < n_chunks // 2 - 1)
            def _():
                start(ci_a + 2, 0, sem_a)

            wait(ci_b, 1, sem_b)
            compute(ci_b, 1)
            return 0

        lax.fori_loop(0, n_chunks // 2, body, 0)
        pltpu.sync_copy(outv, out_hbm.at[pl.ds(base, b_per_w)])

    return sc_call


def kernel(pos_h, pos_r, pos_t, neg_h, neg_r, neg_t, entity_emb, relation_emb):
    b = pos_h.shape[0]
    dim2 = entity_emb.shape[1]
    hidx = jnp.concatenate([pos_h, neg_h]).astype(jnp.int32)
    ridx = jnp.concatenate([pos_r, neg_r]).astype(jnp.int32)
    tidx = jnp.concatenate([pos_t, neg_t]).astype(jnp.int32)
    sc_call = _make_sc_call(2 * b, dim2)
    out = sc_call(hidx, ridx, tidx, entity_emb, relation_emb)
    return out[:b], out[b:]
